# baseline (speedup 1.0000x reference)
import functools

import jax
import jax.numpy as jnp
from jax import lax
from jax.experimental import pallas as pl
from jax.experimental.pallas import tpu as pltpu
from jax.experimental.pallas import tpu_sc as plsc

_N = 1048576
_NC, _NS, _L = 2, 16, 16
_NW = _NC * _NS
_Q = _N // _NW
_C = 4096
_NCHUNK = _Q // _C
_U = 4


def _sc_partials(preds, margin):
    mesh = plsc.VectorSubcoreMesh(core_axis_name="c", subcore_axis_name="s")

    @functools.partial(
        pl.kernel,
        mesh=mesh,
        out_type=jax.ShapeDtypeStruct((_NW, _L), jnp.float32),
        scratch_types=[
            pltpu.VMEM((2, _C), jnp.float32),
            pltpu.VMEM((2, _C), jnp.float32),
            pltpu.VMEM((_L,), jnp.float32),
            pltpu.SemaphoreType.DMA,
            pltpu.SemaphoreType.DMA,
            pltpu.SemaphoreType.DMA,
            pltpu.SemaphoreType.DMA,
        ],
    )
    def k(p_hbm, m_hbm, o_hbm, pbuf, mbuf, accv, sp0, sp1, sm0, sm1):
        wid = lax.axis_index("s") * _NC + lax.axis_index("c")
        base = wid * _Q
        sp = (sp0, sp1)
        sm = (sm0, sm1)

        def issue(ci):
            slot = ci % 2
            off = base + ci * _C
            pltpu.async_copy(p_hbm.at[pl.ds(off, _C)], pbuf.at[slot], sp[slot])
            pltpu.async_copy(m_hbm.at[pl.ds(off, _C)], mbuf.at[slot], sm[slot])

        def wait(ci):
            slot = ci % 2
            off = base + ci * _C
            pltpu.make_async_copy(
                p_hbm.at[pl.ds(off, _C)], pbuf.at[slot], sp[slot]).wait()
            pltpu.make_async_copy(
                m_hbm.at[pl.ds(off, _C)], mbuf.at[slot], sm[slot]).wait()

        issue(0)
        a = jnp.zeros((_L,), jnp.float32)
        for ci in range(_NCHUNK):
            if ci + 1 < _NCHUNK:
                issue(ci + 1)
            wait(ci)
            slot = ci % 2
            pb = pbuf.at[slot]
            mb = mbuf.at[slot]

            def body(j, a):
                for u in range(_U):
                    off = (j * _U + u) * _L
                    pv = pb[pl.ds(off, _L)]
                    mv = mb[pl.ds(off, _L)]
                    w = jnp.exp(mv * mv * -0.5)
                    pm = jnp.where(mv == 0.0, 0.0, pv)
                    a = a + w * pm
                return a

            a = lax.fori_loop(0, _C // (_L * _U), body, a)

        accv[...] = a
        pltpu.sync_copy(accv, o_hbm.at[wid])

    return k(preds, margin)


def kernel(preds, margin):
    partials = _sc_partials(preds, margin)
    return -jnp.sum(partials) / _N


# speedup vs baseline: 6.4879x; 6.4879x over previous
import jax
import jax.numpy as jnp
from jax.experimental import pallas as pl
from jax.experimental.pallas import tpu as pltpu

_N = 1048576
_ROWS = _N // 128
_CROWS = 512
_NCHUNK = _ROWS // _CROWS


def _tc_body(p_hbm, m_hbm, o_ref, pbuf, mbuf, psem, msem):
    for c in range(_NCHUNK):
        pltpu.make_async_copy(
            p_hbm.at[pl.ds(c * _CROWS, _CROWS), :], pbuf.at[c], psem.at[c]
        ).start()
        pltpu.make_async_copy(
            m_hbm.at[pl.ds(c * _CROWS, _CROWS), :], mbuf.at[c], msem.at[c]
        ).start()

    acc = None
    for c in range(_NCHUNK):
        pltpu.make_async_copy(
            p_hbm.at[pl.ds(c * _CROWS, _CROWS), :], pbuf.at[c], psem.at[c]
        ).wait()
        pltpu.make_async_copy(
            m_hbm.at[pl.ds(c * _CROWS, _CROWS), :], mbuf.at[c], msem.at[c]
        ).wait()
        for k in range(0, _CROWS, 64):
            m = mbuf[c, pl.ds(k, 64), :]
            p = pbuf[c, pl.ds(k, 64), :]
            pm = jnp.where(m != 0.0, p, 0.0)
            contrib = jnp.exp(-0.5 * m * m) * pm
            acc = contrib if acc is None else acc + contrib

    while acc.shape[0] > 8:
        h = acc.shape[0] // 2
        acc = acc[:h] + acc[h:]
    o_ref[0, 0] = jnp.sum(acc) * (-1.0 / _N)


def kernel(preds, margin):
    p2 = preds.reshape(_ROWS, 128)
    m2 = margin.reshape(_ROWS, 128)
    out = pl.pallas_call(
        _tc_body,
        in_specs=[
            pl.BlockSpec(memory_space=pl.ANY),
            pl.BlockSpec(memory_space=pl.ANY),
        ],
        out_specs=pl.BlockSpec(memory_space=pltpu.SMEM),
        out_shape=jax.ShapeDtypeStruct((1, 1), jnp.float32),
        scratch_shapes=[
            pltpu.VMEM((_NCHUNK, _CROWS, 128), jnp.float32),
            pltpu.VMEM((_NCHUNK, _CROWS, 128), jnp.float32),
            pltpu.SemaphoreType.DMA((_NCHUNK,)),
            pltpu.SemaphoreType.DMA((_NCHUNK,)),
        ],
    )(p2, m2)
    return out[0, 0]
